# baseline (speedup 1.0000x reference)
import jax
import jax.numpy as jnp
from jax.experimental import pallas as pl

N = 10000
N_HOP = 2
GENERAL_C = 1.0


def _log_softmax_body(x_ref, o_ref):
    x = x_ref[...]
    m = jnp.max(x, axis=1, keepdims=True)
    e = jnp.exp(x - m)
    s = jnp.sum(e, axis=1, keepdims=True)
    o_ref[...] = x - m - jnp.log(s)


def _log_softmax(x):
    rows, cols = x.shape
    blk = 400
    return pl.pallas_call(
        _log_softmax_body,
        grid=(rows // blk,),
        in_specs=[pl.BlockSpec((blk, cols), lambda i: (i, 0))],
        out_specs=pl.BlockSpec((blk, cols), lambda i: (i, 0)),
        out_shape=jax.ShapeDtypeStruct((rows, cols), x.dtype),
    )(x)


def kernel(features, edge_index, W1, b1, W2, b2, W3, b3, Wf):
    h = jax.nn.relu(features @ W1.T + b1)
    h = jax.nn.relu(h @ W2.T + b2)
    h = h @ W3.T + b3

    src = edge_index[0]
    dst = edge_index[1]
    E = src.shape[0]

    deg = jax.ops.segment_sum(jnp.ones((E,), dtype=jnp.float32), dst, num_segments=N)
    deg_c = jnp.maximum(deg, 1.0)

    mean = jax.ops.segment_sum(h[src], dst, num_segments=N) / deg_c[:, None]
    e = mean[dst] - h[src]
    e = e * e
    var = jax.ops.segment_sum(e, dst, num_segments=N) / deg_c[:, None]

    c = GENERAL_C * jax.nn.sigmoid(var @ Wf.T)

    ee = c[src] + c[dst]
    c_sum = jax.ops.segment_sum(ee, dst, num_segments=N)

    de_norm = (1.0 / deg_c)[:, None]
    b = 1.0 / (2.0 + c_sum * de_norm)
    norm = jnp.power(deg_c, -0.5)[:, None]

    feat0 = h
    feat = h
    for _ in range(N_HOP):
        feat = feat * norm
        m = feat[src] * ee
        feat = jax.ops.segment_sum(m, dst, num_segments=N)
        feat = feat * norm
        feat = b * feat + 2.0 * b * feat0

    return _log_softmax(feat)


# speedup vs baseline: 1.7219x; 1.7219x over previous
import functools

import jax
import jax.numpy as jnp
from jax import lax
from jax.experimental import pallas as pl
from jax.experimental.pallas import tpu as pltpu
from jax.experimental.pallas import tpu_sc as plsc

N = 10000
D = 256
DH = 128
DQ = 64
E = 160000
N_HOP = 2

NC = 2
NS = 16
CHUNK = 128
EPT = 10240
NCHUNK = EPT // CHUNK
E_PAD = EPT * NS
NPAD = 10240
RPT = NPAD // NS
RCH = RPT // CHUNK

_mesh = lambda: plsc.VectorSubcoreMesh(core_axis_name="c", subcore_axis_name="s")


def _zero_acc(acc, zb, s):
    base = s * RPT
    for k in range(RCH):
        pltpu.sync_copy(zb, acc.at[pl.ds(base + k * CHUNK, CHUNK)])


def _drain_q(acc, out_a, out_b, bounce, c, s):
    base = s * RPT
    for k in range(RCH):
        sl = pl.ds(base + k * CHUNK, CHUNK)
        pltpu.sync_copy(acc.at[sl], bounce)

        @pl.when(c == 0)
        def _():
            pltpu.sync_copy(bounce, out_a.at[sl])

        @pl.when(c == 1)
        def _():
            pltpu.sync_copy(bounce, out_b.at[sl])


def _edge_pass_q(tbl_qcat, acc, src_hbm, dst_hbm, sidx, sidx2, didx, rows,
                 sem, c, s, p):
    shift = (2 * c + p) * N

    def body(j, _):
        ebase = s * EPT + j * CHUNK
        pltpu.sync_copy(src_hbm.at[pl.ds(ebase, CHUNK)], sidx)
        pltpu.sync_copy(dst_hbm.at[pl.ds(ebase, CHUNK)], didx)
        for k in range(CHUNK // 16):
            sidx2[pl.ds(k * 16, 16)] = sidx[pl.ds(k * 16, 16)] + shift
        pltpu.async_copy(tbl_qcat.at[sidx2], rows, sem).wait()
        pltpu.sync_copy(rows, acc.at[didx], add=True)
        return 0

    lax.fori_loop(0, NCHUNK, body, 0)


def _wide_table_pass(tbl_qcat, acc, outs4, srcp, dstp, sidx, sidx2, didx,
                     rows, zb, sem, c, s):
    for p in range(2):
        _zero_acc(acc, zb, s)
        _bar()
        _edge_pass_q(tbl_qcat, acc, srcp, dstp, sidx, sidx2, didx, rows,
                     sem, c, s, p)
        _bar()
        _drain_q(acc, outs4[p], outs4[2 + p], rows, c, s)
        _bar()


def _edge_pass_16(tbl16, acc16, src_hbm, dst_hbm, sidx, didx, rows16,
                  sem, c, s):
    def body(j, _):
        ebase = s * EPT + j * CHUNK
        pltpu.sync_copy(dst_hbm.at[pl.ds(ebase, CHUNK)], didx)
        if tbl16 is not None:
            pltpu.sync_copy(src_hbm.at[pl.ds(ebase, CHUNK)], sidx)
            pltpu.async_copy(tbl16.at[sidx], rows16, sem).wait()
        pltpu.sync_copy(rows16, acc16.at[didx], add=True)
        return 0

    lax.fori_loop(c * (NCHUNK // 2), (c + 1) * (NCHUNK // 2), body, 0)


def _drain16(acc16, out_a, out_b, bounce16, c, s):
    base = s * RPT
    for k in range(RCH):
        sl = pl.ds(base + k * CHUNK, CHUNK)
        pltpu.sync_copy(acc16.at[sl], bounce16)

        @pl.when(c == 0)
        def _():
            pltpu.sync_copy(bounce16, out_a.at[sl])

        @pl.when(c == 1)
        def _():
            pltpu.sync_copy(bounce16, out_b.at[sl])


def _bar():
    plsc.subcore_barrier()


def _q_out():
    return tuple(jax.ShapeDtypeStruct((NPAD, DQ), jnp.float32) for _ in range(4))


def _base_scratch():
    return [
        pltpu.VMEM((CHUNK,), jnp.int32),
        pltpu.VMEM((CHUNK,), jnp.int32),
        pltpu.VMEM((CHUNK,), jnp.int32),
        pltpu.VMEM((CHUNK, DQ), jnp.float32),
        pltpu.VMEM((CHUNK, DQ), jnp.float32),
        pltpu.VMEM_SHARED((NPAD, DQ), jnp.float32),
        pltpu.SemaphoreType.DMA,
    ]


def _scratch16():
    return [
        pltpu.VMEM((CHUNK, 16), jnp.float32),
        pltpu.VMEM((CHUNK, 16), jnp.float32),
        pltpu.VMEM_SHARED((NPAD, 16), jnp.float32),
    ]


def _build_sc1():
    out_type = _q_out() + _q_out() + (
        jax.ShapeDtypeStruct((NPAD, 16), jnp.float32),
        jax.ShapeDtypeStruct((NPAD, 16), jnp.float32),
    )
    scratch = _base_scratch() + _scratch16() + [
        pltpu.VMEM((CHUNK, 16), jnp.float32),
    ]

    def body(srcp, dstp, z64, o16, z16, h_qcat, h2_qcat,
             s1q0, s1q1, s1q2, s1q3, s2q0, s2q1, s2q2, s2q3, dga, dgb,
             sidx, sidx2, didx, rows, zb, acc, sem, rows16, zb16, acc16, ob16):
        c = lax.axis_index("c")
        s = lax.axis_index("s")
        pltpu.sync_copy(z64, zb)
        pltpu.sync_copy(z16, zb16)
        pltpu.sync_copy(o16, ob16)

        _zero_acc(acc16, zb16, s)
        _bar()
        _edge_pass_16(None, acc16, srcp, dstp, sidx, didx, ob16, sem, c, s)
        _bar()
        _drain16(acc16, dga, dgb, rows16, c, s)
        _bar()

        _wide_table_pass(h_qcat, acc, (s1q0, s1q1, s1q2, s1q3), srcp, dstp,
                         sidx, sidx2, didx, rows, zb, sem, c, s)
        _wide_table_pass(h2_qcat, acc, (s2q0, s2q1, s2q2, s2q3), srcp, dstp,
                         sidx, sidx2, didx, rows, zb, sem, c, s)

    return pl.kernel(body, out_type=out_type, mesh=_mesh(),
                     scratch_types=scratch,
                     compiler_params=pltpu.CompilerParams(
                         use_tc_tiling_on_sc=False))


def _build_sc2():
    out_type = _q_out() + _q_out() + (
        jax.ShapeDtypeStruct((NPAD, 16), jnp.float32),
        jax.ShapeDtypeStruct((NPAD, 16), jnp.float32),
    )
    scratch = _base_scratch() + _scratch16()

    def body(srcp, dstp, z64, z16, c16_tbl, u_qcat, v_qcat,
             aq0, aq1, aq2, aq3, bq0, bq1, bq2, bq3, csa, csb,
             sidx, sidx2, didx, rows, zb, acc, sem, rows16, zb16, acc16):
        c = lax.axis_index("c")
        s = lax.axis_index("s")
        pltpu.sync_copy(z64, zb)
        pltpu.sync_copy(z16, zb16)

        _zero_acc(acc16, zb16, s)
        _bar()
        _edge_pass_16(c16_tbl, acc16, srcp, dstp, sidx, didx, rows16,
                      sem, c, s)
        _bar()
        _drain16(acc16, csa, csb, rows16, c, s)
        _bar()

        _wide_table_pass(u_qcat, acc, (aq0, aq1, aq2, aq3), srcp, dstp,
                         sidx, sidx2, didx, rows, zb, sem, c, s)
        _wide_table_pass(v_qcat, acc, (bq0, bq1, bq2, bq3), srcp, dstp,
                         sidx, sidx2, didx, rows, zb, sem, c, s)

    return pl.kernel(body, out_type=out_type, mesh=_mesh(),
                     scratch_types=scratch,
                     compiler_params=pltpu.CompilerParams(
                         use_tc_tiling_on_sc=False))


def _build_sc3():
    out_type = _q_out() + _q_out()
    scratch = _base_scratch()

    def body(srcp, dstp, z64, u_qcat, v_qcat,
             aq0, aq1, aq2, aq3, bq0, bq1, bq2, bq3,
             sidx, sidx2, didx, rows, zb, acc, sem):
        c = lax.axis_index("c")
        s = lax.axis_index("s")
        pltpu.sync_copy(z64, zb)

        _wide_table_pass(u_qcat, acc, (aq0, aq1, aq2, aq3), srcp, dstp,
                         sidx, sidx2, didx, rows, zb, sem, c, s)
        _wide_table_pass(v_qcat, acc, (bq0, bq1, bq2, bq3), srcp, dstp,
                         sidx, sidx2, didx, rows, zb, sem, c, s)

    return pl.kernel(body, out_type=out_type, mesh=_mesh(),
                     scratch_types=scratch,
                     compiler_params=pltpu.CompilerParams(
                         use_tc_tiling_on_sc=False))


_R = 400
_G = N // _R


def _mlp_body(x_ref, w1_ref, b1_ref, w2_ref, b2_ref, w3_ref, b3_ref,
              h_ref, hsq_ref):
    x = x_ref[...]
    h1 = jnp.maximum(
        jnp.dot(x, w1_ref[...], preferred_element_type=jnp.float32)
        + b1_ref[...], 0.0)
    h2 = jnp.maximum(
        jnp.dot(h1, w2_ref[...], preferred_element_type=jnp.float32)
        + b2_ref[...], 0.0)
    h = jnp.dot(h2, w3_ref[...], preferred_element_type=jnp.float32) + b3_ref[...]
    h_ref[...] = h
    hsq_ref[...] = h * h


def _mlp_call(x, w1t, b1r, w2t, b2r, w3t, b3r):
    full = lambda shp: pl.BlockSpec(shp, lambda i: tuple(0 for _ in shp))
    return pl.pallas_call(
        _mlp_body,
        grid=(_G,),
        in_specs=[
            pl.BlockSpec((_R, D), lambda i: (i, 0)),
            full((D, 512)), full((1, 512)),
            full((512, 512)), full((1, 512)),
            full((512, D)), full((1, D)),
        ],
        out_specs=[pl.BlockSpec((_R, D), lambda i: (i, 0))] * 2,
        out_shape=[jax.ShapeDtypeStruct((N, D), jnp.float32)] * 2,
    )(x, w1t, b1r, w2t, b2r, w3t, b3r)


def _stage2_body(s1_ref, s2_ref, dg_ref, h_ref, wf_ref,
                 c16_ref, u1_ref, v1_ref):
    deg = dg_ref[...][:, :1]
    deg_c = jnp.maximum(deg, 1.0)
    s1 = s1_ref[...]
    mean = s1 / deg_c
    var = (s2_ref[...] - 2.0 * mean * s1 + deg * mean * mean) / deg_c
    logit = jnp.sum(var * wf_ref[...], axis=1, keepdims=True)
    cval = jax.nn.sigmoid(logit)
    norm = lax.rsqrt(deg_c)
    u1 = h_ref[...] * norm
    c16_ref[...] = jnp.broadcast_to(cval, (_R, 16))
    u1_ref[...] = u1
    v1_ref[...] = u1 * cval


def _stage2_call(s1, s2, dg16, h, wf):
    full = lambda shp: pl.BlockSpec(shp, lambda i: tuple(0 for _ in shp))
    blkD = pl.BlockSpec((_R, D), lambda i: (i, 0))
    blk16 = pl.BlockSpec((_R, 16), lambda i: (i, 0))
    return pl.pallas_call(
        _stage2_body,
        grid=(_G,),
        in_specs=[blkD, blkD, blk16, blkD, full((1, D))],
        out_specs=[blk16, blkD, blkD],
        out_shape=[
            jax.ShapeDtypeStruct((N, 16), jnp.float32),
            jax.ShapeDtypeStruct((N, D), jnp.float32),
            jax.ShapeDtypeStruct((N, D), jnp.float32),
        ],
    )(s1, s2, dg16, h, wf)


def _hop_combine(a_ref, b_ref, cs_ref, dg_ref, c16_ref, h_ref, *out_refs):
    deg = dg_ref[...][:, :1]
    deg_c = jnp.maximum(deg, 1.0)
    norm = lax.rsqrt(deg_c)
    cval = c16_ref[...][:, :1]
    c_sum = cs_ref[...][:, :1] + deg * cval
    bb = 1.0 / (2.0 + c_sum / deg_c)
    feat = bb * ((b_ref[...] + cval * a_ref[...]) * norm) + 2.0 * bb * h_ref[...]
    return feat, norm, cval, out_refs


def _stage3_body(a_ref, b_ref, cs_ref, dg_ref, c16_ref, h_ref,
                 u2_ref, v2_ref):
    feat, norm, cval, _ = _hop_combine(a_ref, b_ref, cs_ref, dg_ref, c16_ref,
                                       h_ref)
    u2 = feat * norm
    u2_ref[...] = u2
    v2_ref[...] = u2 * cval


def _stage4_body(a_ref, b_ref, cs_ref, dg_ref, c16_ref, h_ref, out_ref):
    feat, _, _, _ = _hop_combine(a_ref, b_ref, cs_ref, dg_ref, c16_ref, h_ref)
    m = jnp.max(feat, axis=1, keepdims=True)
    ex = jnp.exp(feat - m)
    out_ref[...] = feat - m - jnp.log(jnp.sum(ex, axis=1, keepdims=True))


def _stage34_call(body, n_out, a, b, cs16, dg16, c16, h):
    blkD = pl.BlockSpec((_R, D), lambda i: (i, 0))
    blk16 = pl.BlockSpec((_R, 16), lambda i: (i, 0))
    out_shape = [jax.ShapeDtypeStruct((N, D), jnp.float32)] * n_out
    out_specs = [blkD] * n_out
    if n_out == 1:
        out_shape, out_specs = out_shape[0], out_specs[0]
    return pl.pallas_call(
        body,
        grid=(_G,),
        in_specs=[blkD, blkD, blk16, blk16, blk16, blkD],
        out_specs=out_specs,
        out_shape=out_shape,
    )(a, b, cs16, dg16, c16, h)


def kernel(features, edge_index, W1, b1, W2, b2, W3, b3, Wf):
    src = edge_index[0].astype(jnp.int32)
    dst = edge_index[1].astype(jnp.int32)
    pad = E_PAD - E
    srcp = jnp.concatenate([src, jnp.zeros((pad,), jnp.int32)])
    dstp = jnp.concatenate([dst, jnp.full((pad,), N, jnp.int32)])

    z64 = jnp.zeros((CHUNK, DQ), jnp.float32)
    z16 = jnp.zeros((CHUNK, 16), jnp.float32)
    o16 = jnp.ones((CHUNK, 16), jnp.float32)

    h, hsq = _mlp_call(features, W1.T, b1.reshape(1, -1), W2.T,
                       b2.reshape(1, -1), W3.T, b3.reshape(1, -1))

    qcat = lambda t: jnp.concatenate(
        [t[:, k * DQ:(k + 1) * DQ] for k in range(4)], axis=0)
    uncat = lambda qs: jnp.concatenate([q[:N] for q in qs], axis=1)

    sc1 = _build_sc1()
    (s1q0, s1q1, s1q2, s1q3, s2q0, s2q1, s2q2, s2q3, dga, dgb) = sc1(
        srcp, dstp, z64, o16, z16, qcat(h), qcat(hsq))
    dg16 = (dga + dgb)[:N]
    s1 = uncat((s1q0, s1q1, s1q2, s1q3))
    s2 = uncat((s2q0, s2q1, s2q2, s2q3))

    c16, u1, v1 = _stage2_call(s1, s2, dg16, h, Wf.reshape(1, -1))

    sc2 = _build_sc2()
    (aq0, aq1, aq2, aq3, bq0, bq1, bq2, bq3, csa, csb) = sc2(
        srcp, dstp, z64, z16, c16, qcat(u1), qcat(v1))
    cs16 = (csa + csb)[:N]
    a1 = uncat((aq0, aq1, aq2, aq3))
    b1v = uncat((bq0, bq1, bq2, bq3))

    u2, v2 = _stage34_call(_stage3_body, 2, a1, b1v, cs16, dg16, c16, h)

    sc3 = _build_sc3()
    (cq0, cq1, cq2, cq3, dq0, dq1, dq2, dq3) = sc3(
        srcp, dstp, z64, qcat(u2), qcat(v2))
    a2 = uncat((cq0, cq1, cq2, cq3))
    b2v = uncat((dq0, dq1, dq2, dq3))

    return _stage34_call(_stage4_body, 1, a2, b2v, cs16, dg16, c16, h)


# speedup vs baseline: 2.4150x; 1.4026x over previous
import functools

import jax
import jax.numpy as jnp
from jax import lax
from jax.experimental import pallas as pl
from jax.experimental.pallas import tpu as pltpu
from jax.experimental.pallas import tpu_sc as plsc

N = 10000
D = 256
DH = 128
DQ = 64
E = 160000
N_HOP = 2

NC = 2
NS = 16
CHUNK = 128
EPT = 10240
NCHUNK = EPT // CHUNK
E_PAD = EPT * NS
NPAD = 10240
RPT = NPAD // NS
RCH = RPT // CHUNK

_mesh = lambda: plsc.VectorSubcoreMesh(core_axis_name="c", subcore_axis_name="s")


def _zero_acc(acc, zb, s):
    base = s * RPT
    for k in range(RCH):
        pltpu.sync_copy(zb, acc.at[pl.ds(base + k * CHUNK, CHUNK)])


def _drain_q(acc, out_a, out_b, bounce, c, s):
    base = s * RPT
    for k in range(RCH):
        sl = pl.ds(base + k * CHUNK, CHUNK)
        pltpu.sync_copy(acc.at[sl], bounce)

        @pl.when(c == 0)
        def _():
            pltpu.sync_copy(bounce, out_a.at[sl])

        @pl.when(c == 1)
        def _():
            pltpu.sync_copy(bounce, out_b.at[sl])


def _load_tile_idx(srcp3, dstp3, src2d, dst2d, s):
    pltpu.sync_copy(srcp3.at[s], src2d)
    pltpu.sync_copy(dstp3.at[s], dst2d)


def _shift_idx(src2d, out2d, shift):
    def body(j, _):
        for k in range(CHUNK // 16):
            out2d[j, pl.ds(k * 16, 16)] = src2d[j, pl.ds(k * 16, 16)] + shift
        return 0

    lax.fori_loop(0, NCHUNK, body, 0)


def _edge_pass_q(tbl, acc, sidx2d, dst2d, rowsA, rowsB, semA, semB):
    pltpu.async_copy(tbl.at[sidx2d.at[0]], rowsA, semA)

    def body(t, _):
        j0 = 2 * t
        pltpu.make_async_copy(tbl.at[sidx2d.at[0]], rowsA, semA).wait()
        pltpu.async_copy(tbl.at[sidx2d.at[j0 + 1]], rowsB, semB)
        pltpu.sync_copy(rowsA, acc.at[dst2d.at[j0]], add=True)
        pltpu.make_async_copy(tbl.at[sidx2d.at[0]], rowsB, semB).wait()

        @pl.when(t < NCHUNK // 2 - 1)
        def _():
            pltpu.async_copy(tbl.at[sidx2d.at[j0 + 2]], rowsA, semA)

        pltpu.sync_copy(rowsB, acc.at[dst2d.at[j0 + 1]], add=True)
        return 0

    lax.fori_loop(0, NCHUNK // 2, body, 0)


def _wide_table_pass(tbl_qcat, acc, outs4, shifted, dst2d, rowsA, rowsB,
                     zb, semA, semB, c, s):
    for p in range(2):
        _zero_acc(acc, zb, s)
        _bar()
        _edge_pass_q(tbl_qcat, acc, shifted[p], dst2d, rowsA, rowsB,
                     semA, semB)
        _bar()
        _drain_q(acc, outs4[p], outs4[2 + p], rowsA, c, s)
        _bar()


def _edge_pass_16(tbl16, acc16, src2d, dst2d, rows16, sem, c):
    def body(j, _):
        if tbl16 is not None:
            pltpu.async_copy(tbl16.at[src2d.at[j]], rows16, sem).wait()
        pltpu.sync_copy(rows16, acc16.at[dst2d.at[j]], add=True)
        return 0

    lax.fori_loop(c * (NCHUNK // 2), (c + 1) * (NCHUNK // 2), body, 0)


def _drain16(acc16, out_a, out_b, bounce16, c, s):
    base = s * RPT
    for k in range(RCH):
        sl = pl.ds(base + k * CHUNK, CHUNK)
        pltpu.sync_copy(acc16.at[sl], bounce16)

        @pl.when(c == 0)
        def _():
            pltpu.sync_copy(bounce16, out_a.at[sl])

        @pl.when(c == 1)
        def _():
            pltpu.sync_copy(bounce16, out_b.at[sl])


def _bar():
    plsc.subcore_barrier()


def _q_out():
    return tuple(jax.ShapeDtypeStruct((NPAD, DQ), jnp.float32) for _ in range(4))


def _base_scratch():
    return [
        pltpu.VMEM((NCHUNK, CHUNK), jnp.int32),
        pltpu.VMEM((NCHUNK, CHUNK), jnp.int32),
        pltpu.VMEM((NCHUNK, CHUNK), jnp.int32),
        pltpu.VMEM((NCHUNK, CHUNK), jnp.int32),
        pltpu.VMEM((CHUNK, DQ), jnp.float32),
        pltpu.VMEM((CHUNK, DQ), jnp.float32),
        pltpu.VMEM((CHUNK, DQ), jnp.float32),
        pltpu.VMEM_SHARED((NPAD, DQ), jnp.float32),
        pltpu.SemaphoreType.DMA,
        pltpu.SemaphoreType.DMA,
    ]


def _scratch16():
    return [
        pltpu.VMEM((CHUNK, 16), jnp.float32),
        pltpu.VMEM((CHUNK, 16), jnp.float32),
        pltpu.VMEM_SHARED((NPAD, 16), jnp.float32),
    ]


def _prolog(srcp3, dstp3, src2d, dst2d, srcsh0, srcsh1, c, s):
    _load_tile_idx(srcp3, dstp3, src2d, dst2d, s)
    _shift_idx(src2d, srcsh0, (2 * c) * N)
    _shift_idx(src2d, srcsh1, (2 * c + 1) * N)


def _build_sc1():
    out_type = _q_out() + _q_out() + (
        jax.ShapeDtypeStruct((NPAD, 16), jnp.float32),
        jax.ShapeDtypeStruct((NPAD, 16), jnp.float32),
    )
    scratch = _base_scratch() + _scratch16() + [
        pltpu.VMEM((CHUNK, 16), jnp.float32),
    ]

    def body(srcp3, dstp3, z64, o16, z16, h_qcat, h2_qcat,
             s1q0, s1q1, s1q2, s1q3, s2q0, s2q1, s2q2, s2q3, dga, dgb,
             src2d, dst2d, srcsh0, srcsh1, rowsA, rowsB, zb, acc, semA, semB,
             rows16, zb16, acc16, ob16):
        c = lax.axis_index("c")
        s = lax.axis_index("s")
        pltpu.sync_copy(z64, zb)
        pltpu.sync_copy(z16, zb16)
        pltpu.sync_copy(o16, ob16)
        _prolog(srcp3, dstp3, src2d, dst2d, srcsh0, srcsh1, c, s)

        _zero_acc(acc16, zb16, s)
        _bar()
        _edge_pass_16(None, acc16, src2d, dst2d, ob16, semA, c)
        _bar()
        _drain16(acc16, dga, dgb, rows16, c, s)
        _bar()

        _wide_table_pass(h_qcat, acc, (s1q0, s1q1, s1q2, s1q3),
                         (srcsh0, srcsh1), dst2d, rowsA, rowsB, zb,
                         semA, semB, c, s)
        _wide_table_pass(h2_qcat, acc, (s2q0, s2q1, s2q2, s2q3),
                         (srcsh0, srcsh1), dst2d, rowsA, rowsB, zb,
                         semA, semB, c, s)

    return pl.kernel(body, out_type=out_type, mesh=_mesh(),
                     scratch_types=scratch,
                     compiler_params=pltpu.CompilerParams(
                         use_tc_tiling_on_sc=False))


def _build_sc2():
    out_type = _q_out() + _q_out() + (
        jax.ShapeDtypeStruct((NPAD, 16), jnp.float32),
        jax.ShapeDtypeStruct((NPAD, 16), jnp.float32),
    )
    scratch = _base_scratch() + _scratch16()

    def body(srcp3, dstp3, z64, z16, c16_tbl, u_qcat, v_qcat,
             aq0, aq1, aq2, aq3, bq0, bq1, bq2, bq3, csa, csb,
             src2d, dst2d, srcsh0, srcsh1, rowsA, rowsB, zb, acc, semA, semB,
             rows16, zb16, acc16):
        c = lax.axis_index("c")
        s = lax.axis_index("s")
        pltpu.sync_copy(z64, zb)
        pltpu.sync_copy(z16, zb16)
        _prolog(srcp3, dstp3, src2d, dst2d, srcsh0, srcsh1, c, s)

        _zero_acc(acc16, zb16, s)
        _bar()
        _edge_pass_16(c16_tbl, acc16, src2d, dst2d, rows16, semA, c)
        _bar()
        _drain16(acc16, csa, csb, rows16, c, s)
        _bar()

        _wide_table_pass(u_qcat, acc, (aq0, aq1, aq2, aq3),
                         (srcsh0, srcsh1), dst2d, rowsA, rowsB, zb,
                         semA, semB, c, s)
        _wide_table_pass(v_qcat, acc, (bq0, bq1, bq2, bq3),
                         (srcsh0, srcsh1), dst2d, rowsA, rowsB, zb,
                         semA, semB, c, s)

    return pl.kernel(body, out_type=out_type, mesh=_mesh(),
                     scratch_types=scratch,
                     compiler_params=pltpu.CompilerParams(
                         use_tc_tiling_on_sc=False))


def _build_sc3():
    out_type = _q_out() + _q_out()
    scratch = _base_scratch()

    def body(srcp3, dstp3, z64, u_qcat, v_qcat,
             aq0, aq1, aq2, aq3, bq0, bq1, bq2, bq3,
             src2d, dst2d, srcsh0, srcsh1, rowsA, rowsB, zb, acc, semA, semB):
        c = lax.axis_index("c")
        s = lax.axis_index("s")
        pltpu.sync_copy(z64, zb)
        _prolog(srcp3, dstp3, src2d, dst2d, srcsh0, srcsh1, c, s)

        _wide_table_pass(u_qcat, acc, (aq0, aq1, aq2, aq3),
                         (srcsh0, srcsh1), dst2d, rowsA, rowsB, zb,
                         semA, semB, c, s)
        _wide_table_pass(v_qcat, acc, (bq0, bq1, bq2, bq3),
                         (srcsh0, srcsh1), dst2d, rowsA, rowsB, zb,
                         semA, semB, c, s)

    return pl.kernel(body, out_type=out_type, mesh=_mesh(),
                     scratch_types=scratch,
                     compiler_params=pltpu.CompilerParams(
                         use_tc_tiling_on_sc=False))


_R = 400
_G = N // _R


def _mlp_body(x_ref, w1_ref, b1_ref, w2_ref, b2_ref, w3_ref, b3_ref,
              h_ref, hsq_ref):
    x = x_ref[...]
    h1 = jnp.maximum(
        jnp.dot(x, w1_ref[...], preferred_element_type=jnp.float32)
        + b1_ref[...], 0.0)
    h2 = jnp.maximum(
        jnp.dot(h1, w2_ref[...], preferred_element_type=jnp.float32)
        + b2_ref[...], 0.0)
    h = jnp.dot(h2, w3_ref[...], preferred_element_type=jnp.float32) + b3_ref[...]
    h_ref[...] = h
    hsq_ref[...] = h * h


def _mlp_call(x, w1t, b1r, w2t, b2r, w3t, b3r):
    full = lambda shp: pl.BlockSpec(shp, lambda i: tuple(0 for _ in shp))
    return pl.pallas_call(
        _mlp_body,
        grid=(_G,),
        in_specs=[
            pl.BlockSpec((_R, D), lambda i: (i, 0)),
            full((D, 512)), full((1, 512)),
            full((512, 512)), full((1, 512)),
            full((512, D)), full((1, D)),
        ],
        out_specs=[pl.BlockSpec((_R, D), lambda i: (i, 0))] * 2,
        out_shape=[jax.ShapeDtypeStruct((N, D), jnp.float32)] * 2,
    )(x, w1t, b1r, w2t, b2r, w3t, b3r)


def _stage2_body(s1_ref, s2_ref, dg_ref, h_ref, wf_ref,
                 c16_ref, u1_ref, v1_ref):
    deg = dg_ref[...][:, :1]
    deg_c = jnp.maximum(deg, 1.0)
    s1 = s1_ref[...]
    mean = s1 / deg_c
    var = (s2_ref[...] - 2.0 * mean * s1 + deg * mean * mean) / deg_c
    logit = jnp.sum(var * wf_ref[...], axis=1, keepdims=True)
    cval = jax.nn.sigmoid(logit)
    norm = lax.rsqrt(deg_c)
    u1 = h_ref[...] * norm
    c16_ref[...] = jnp.broadcast_to(cval, (_R, 16))
    u1_ref[...] = u1
    v1_ref[...] = u1 * cval


def _stage2_call(s1, s2, dg16, h, wf):
    full = lambda shp: pl.BlockSpec(shp, lambda i: tuple(0 for _ in shp))
    blkD = pl.BlockSpec((_R, D), lambda i: (i, 0))
    blk16 = pl.BlockSpec((_R, 16), lambda i: (i, 0))
    return pl.pallas_call(
        _stage2_body,
        grid=(_G,),
        in_specs=[blkD, blkD, blk16, blkD, full((1, D))],
        out_specs=[blk16, blkD, blkD],
        out_shape=[
            jax.ShapeDtypeStruct((N, 16), jnp.float32),
            jax.ShapeDtypeStruct((N, D), jnp.float32),
            jax.ShapeDtypeStruct((N, D), jnp.float32),
        ],
    )(s1, s2, dg16, h, wf)


def _hop_combine(a_ref, b_ref, cs_ref, dg_ref, c16_ref, h_ref, *out_refs):
    deg = dg_ref[...][:, :1]
    deg_c = jnp.maximum(deg, 1.0)
    norm = lax.rsqrt(deg_c)
    cval = c16_ref[...][:, :1]
    c_sum = cs_ref[...][:, :1] + deg * cval
    bb = 1.0 / (2.0 + c_sum / deg_c)
    feat = bb * ((b_ref[...] + cval * a_ref[...]) * norm) + 2.0 * bb * h_ref[...]
    return feat, norm, cval, out_refs


def _stage3_body(a_ref, b_ref, cs_ref, dg_ref, c16_ref, h_ref,
                 u2_ref, v2_ref):
    feat, norm, cval, _ = _hop_combine(a_ref, b_ref, cs_ref, dg_ref, c16_ref,
                                       h_ref)
    u2 = feat * norm
    u2_ref[...] = u2
    v2_ref[...] = u2 * cval


def _stage4_body(a_ref, b_ref, cs_ref, dg_ref, c16_ref, h_ref, out_ref):
    feat, _, _, _ = _hop_combine(a_ref, b_ref, cs_ref, dg_ref, c16_ref, h_ref)
    m = jnp.max(feat, axis=1, keepdims=True)
    ex = jnp.exp(feat - m)
    out_ref[...] = feat - m - jnp.log(jnp.sum(ex, axis=1, keepdims=True))


def _stage34_call(body, n_out, a, b, cs16, dg16, c16, h):
    blkD = pl.BlockSpec((_R, D), lambda i: (i, 0))
    blk16 = pl.BlockSpec((_R, 16), lambda i: (i, 0))
    out_shape = [jax.ShapeDtypeStruct((N, D), jnp.float32)] * n_out
    out_specs = [blkD] * n_out
    if n_out == 1:
        out_shape, out_specs = out_shape[0], out_specs[0]
    return pl.pallas_call(
        body,
        grid=(_G,),
        in_specs=[blkD, blkD, blk16, blk16, blk16, blkD],
        out_specs=out_specs,
        out_shape=out_shape,
    )(a, b, cs16, dg16, c16, h)


def kernel(features, edge_index, W1, b1, W2, b2, W3, b3, Wf):
    src = edge_index[0].astype(jnp.int32)
    dst = edge_index[1].astype(jnp.int32)
    pad = E_PAD - E
    srcp = jnp.concatenate([src, jnp.zeros((pad,), jnp.int32)]).reshape(
        NS, NCHUNK, CHUNK)
    dstp = jnp.concatenate([dst, jnp.full((pad,), N, jnp.int32)]).reshape(
        NS, NCHUNK, CHUNK)

    z64 = jnp.zeros((CHUNK, DQ), jnp.float32)
    z16 = jnp.zeros((CHUNK, 16), jnp.float32)
    o16 = jnp.ones((CHUNK, 16), jnp.float32)

    h, hsq = _mlp_call(features, W1.T, b1.reshape(1, -1), W2.T,
                       b2.reshape(1, -1), W3.T, b3.reshape(1, -1))

    qcat = lambda t: jnp.concatenate(
        [t[:, k * DQ:(k + 1) * DQ] for k in range(4)], axis=0)
    uncat = lambda qs: jnp.concatenate([q[:N] for q in qs], axis=1)

    sc1 = _build_sc1()
    (s1q0, s1q1, s1q2, s1q3, s2q0, s2q1, s2q2, s2q3, dga, dgb) = sc1(
        srcp, dstp, z64, o16, z16, qcat(h), qcat(hsq))
    dg16 = (dga + dgb)[:N]
    s1 = uncat((s1q0, s1q1, s1q2, s1q3))
    s2 = uncat((s2q0, s2q1, s2q2, s2q3))

    c16, u1, v1 = _stage2_call(s1, s2, dg16, h, Wf.reshape(1, -1))

    sc2 = _build_sc2()
    (aq0, aq1, aq2, aq3, bq0, bq1, bq2, bq3, csa, csb) = sc2(
        srcp, dstp, z64, z16, c16, qcat(u1), qcat(v1))
    cs16 = (csa + csb)[:N]
    a1 = uncat((aq0, aq1, aq2, aq3))
    b1v = uncat((bq0, bq1, bq2, bq3))

    u2, v2 = _stage34_call(_stage3_body, 2, a1, b1v, cs16, dg16, c16, h)

    sc3 = _build_sc3()
    (cq0, cq1, cq2, cq3, dq0, dq1, dq2, dq3) = sc3(
        srcp, dstp, z64, qcat(u2), qcat(v2))
    a2 = uncat((cq0, cq1, cq2, cq3))
    b2v = uncat((dq0, dq1, dq2, dq3))

    return _stage34_call(_stage4_body, 1, a2, b2v, cs16, dg16, c16, h)


# speedup vs baseline: 3.3843x; 1.4013x over previous
import jax
import jax.numpy as jnp
from jax import lax
from jax.experimental import pallas as pl
from jax.experimental.pallas import tpu as pltpu
from jax.experimental.pallas import tpu_sc as plsc

N = 10000
D = 256
DQ = 64
E = 160000

NS = 16
CHUNK = 128
EPT = 10240
NCHUNK = EPT // CHUNK
HCHUNK = NCHUNK // 2
E_PAD = EPT * NS
NPAD = 10240
NP16 = N + 16
RPT = NPAD // NS
RCH = RPT // CHUNK

_mesh = lambda: plsc.VectorSubcoreMesh(core_axis_name="c", subcore_axis_name="s")
_params = lambda: pltpu.CompilerParams(use_tc_tiling_on_sc=False)


def _bar():
    plsc.subcore_barrier()


def _zero_acc(acc, zb, s):
    base = s * RPT
    for k in range(RCH):
        pltpu.sync_copy(zb, acc.at[pl.ds(base + k * CHUNK, CHUNK)])


def _drain_zero(acc, out_a, out_b, bounce, zb, c, s):
    base = s * RPT
    for k in range(RCH):
        sl = pl.ds(base + k * CHUNK, CHUNK)
        pltpu.sync_copy(acc.at[sl], bounce)

        @pl.when(c == 0)
        def _():
            pltpu.sync_copy(bounce, out_a.at[sl])

        @pl.when(c == 1)
        def _():
            pltpu.sync_copy(bounce, out_b.at[sl])

        pltpu.sync_copy(zb, acc.at[sl])


def _prolog(srcp3, dstp3, src2d, dst2d, srcsh0, srcsh1, c, s):
    pltpu.sync_copy(srcp3.at[s], src2d)
    pltpu.sync_copy(dstp3.at[s], dst2d)
    for buf, shift in ((srcsh0, (2 * c) * N), (srcsh1, (2 * c + 1) * N)):

        def body(j, _, buf=buf, shift=shift):
            for k in range(CHUNK // 16):
                buf[j, pl.ds(k * 16, 16)] = src2d[j, pl.ds(k * 16, 16)] + shift
            return 0

        lax.fori_loop(0, NCHUNK, body, 0)


def _plain_pass(tbl, acc, sidx2d, dst2d, rowsA, rowsB, semA, semB,
                fold16=None, c=None):
    pltpu.async_copy(tbl.at[sidx2d.at[0]], rowsA, semA)

    def fold(j):
        if fold16 is not None:
            acc16, ob16 = fold16

            @pl.when(jnp.logical_and(j >= c * HCHUNK, j < (c + 1) * HCHUNK))
            def _():
                pltpu.sync_copy(ob16, acc16.at[dst2d.at[j]], add=True)

    def body(t, _):
        j0 = 2 * t
        pltpu.make_async_copy(tbl.at[sidx2d.at[0]], rowsA, semA).wait()
        pltpu.async_copy(tbl.at[sidx2d.at[j0 + 1]], rowsB, semB)
        fold(j0)
        pltpu.sync_copy(rowsA, acc.at[dst2d.at[j0]], add=True)
        pltpu.make_async_copy(tbl.at[sidx2d.at[0]], rowsB, semB).wait()

        @pl.when(t < HCHUNK - 1)
        def _():
            pltpu.async_copy(tbl.at[sidx2d.at[j0 + 2]], rowsA, semA)

        fold(j0 + 1)
        pltpu.sync_copy(rowsB, acc.at[dst2d.at[j0 + 1]], add=True)
        return 0

    lax.fori_loop(0, HCHUNK, body, 0)


def _hop_issue(tbl, c16p, sidx2d, src2d, dst2d, j, rows, cs, cd, sem):
    pltpu.async_copy(tbl.at[sidx2d.at[j]], rows, sem)
    pltpu.async_copy(c16p.at[src2d.at[j]], cs, sem)
    pltpu.async_copy(c16p.at[dst2d.at[j]], cd, sem)


def _hop_wait(tbl, c16p, sidx2d, src2d, rows, cs, cd, sem):
    pltpu.make_async_copy(tbl.at[sidx2d.at[0]], rows, sem).wait()
    pltpu.make_async_copy(c16p.at[src2d.at[0]], cs, sem).wait()
    pltpu.make_async_copy(c16p.at[src2d.at[0]], cd, sem).wait()


def _scale_rows(rows, cs, cd):
    def rbody(r, _):
        ee = cs[r, pl.ds(0, 16)] + cd[r, pl.ds(0, 16)]
        for k in range(DQ // 16):
            rows[r, pl.ds(k * 16, 16)] = rows[r, pl.ds(k * 16, 16)] * ee
        return 0

    lax.fori_loop(0, CHUNK, rbody, 0)


def _hop_pass(tbl, c16p, acc, sidx2d, src2d, dst2d, rowsA, rowsB,
              csA, csB, cdA, cdB, semA, semB, fold_acc16=None, c=None):
    _hop_issue(tbl, c16p, sidx2d, src2d, dst2d, 0, rowsA, csA, cdA, semA)

    def fold(j, cs):
        if fold_acc16 is not None:

            @pl.when(jnp.logical_and(j >= c * HCHUNK, j < (c + 1) * HCHUNK))
            def _():
                pltpu.sync_copy(cs, fold_acc16.at[dst2d.at[j]], add=True)

    def body(t, _):
        j0 = 2 * t
        _hop_wait(tbl, c16p, sidx2d, src2d, rowsA, csA, cdA, semA)
        _hop_issue(tbl, c16p, sidx2d, src2d, dst2d, j0 + 1, rowsB, csB, cdB,
                   semB)
        fold(j0, csA)
        _scale_rows(rowsA, csA, cdA)
        pltpu.sync_copy(rowsA, acc.at[dst2d.at[j0]], add=True)
        _hop_wait(tbl, c16p, sidx2d, src2d, rowsB, csB, cdB, semB)

        @pl.when(t < HCHUNK - 1)
        def _():
            _hop_issue(tbl, c16p, sidx2d, src2d, dst2d, j0 + 2, rowsA, csA,
                       cdA, semA)

        fold(j0 + 1, csB)
        _scale_rows(rowsB, csB, cdB)
        pltpu.sync_copy(rowsB, acc.at[dst2d.at[j0 + 1]], add=True)
        return 0

    lax.fori_loop(0, HCHUNK, body, 0)


def _drain16_zero(acc16, out_a, out_b, bounce16, zb16, c, s):
    base = s * RPT
    for k in range(RCH):
        sl = pl.ds(base + k * CHUNK, CHUNK)
        pltpu.sync_copy(acc16.at[sl], bounce16)

        @pl.when(c == 0)
        def _():
            pltpu.sync_copy(bounce16, out_a.at[sl])

        @pl.when(c == 1)
        def _():
            pltpu.sync_copy(bounce16, out_b.at[sl])

        pltpu.sync_copy(zb16, acc16.at[sl])


def _q_out():
    return tuple(jax.ShapeDtypeStruct((NPAD, DQ), jnp.float32) for _ in range(4))


def _o16_out():
    return (jax.ShapeDtypeStruct((NPAD, 16), jnp.float32),
            jax.ShapeDtypeStruct((NPAD, 16), jnp.float32))


def _base_scratch():
    return [
        pltpu.VMEM((NCHUNK, CHUNK), jnp.int32),
        pltpu.VMEM((NCHUNK, CHUNK), jnp.int32),
        pltpu.VMEM((NCHUNK, CHUNK), jnp.int32),
        pltpu.VMEM((NCHUNK, CHUNK), jnp.int32),
        pltpu.VMEM((CHUNK, DQ), jnp.float32),
        pltpu.VMEM((CHUNK, DQ), jnp.float32),
        pltpu.VMEM((CHUNK, DQ), jnp.float32),
        pltpu.VMEM_SHARED((NPAD, DQ), jnp.float32),
        pltpu.SemaphoreType.DMA,
        pltpu.SemaphoreType.DMA,
    ]


def _build_sc1():
    out_type = _q_out() + _q_out() + _o16_out()
    scratch = _base_scratch() + [
        pltpu.VMEM((CHUNK, 16), jnp.float32),
        pltpu.VMEM((CHUNK, 16), jnp.float32),
        pltpu.VMEM((CHUNK, 16), jnp.float32),
        pltpu.VMEM_SHARED((NPAD, 16), jnp.float32),
    ]

    def body(srcp3, dstp3, z64, o16, z16, h_qcat, h2_qcat,
             s1q0, s1q1, s1q2, s1q3, s2q0, s2q1, s2q2, s2q3, dga, dgb,
             src2d, dst2d, srcsh0, srcsh1, rowsA, rowsB, zb, acc, semA, semB,
             rows16, zb16, ob16, acc16):
        c = lax.axis_index("c")
        s = lax.axis_index("s")
        pltpu.sync_copy(z64, zb)
        pltpu.sync_copy(z16, zb16)
        pltpu.sync_copy(o16, ob16)
        _prolog(srcp3, dstp3, src2d, dst2d, srcsh0, srcsh1, c, s)
        _zero_acc(acc, zb, s)
        _zero_acc(acc16, zb16, s)
        _bar()

        s1o = ((s1q0, s1q2), (s1q1, s1q3))
        s2o = ((s2q0, s2q2), (s2q1, s2q3))
        for p, sh in enumerate((srcsh0, srcsh1)):
            _plain_pass(h_qcat, acc, sh, dst2d, rowsA, rowsB, semA, semB,
                        fold16=(acc16, ob16) if p == 0 else None, c=c)
            _bar()
            _drain_zero(acc, s1o[p][0], s1o[p][1], rowsA, zb, c, s)
            if p == 0:
                _drain16_zero(acc16, dga, dgb, rows16, zb16, c, s)
            _bar()
        for p, sh in enumerate((srcsh0, srcsh1)):
            _plain_pass(h2_qcat, acc, sh, dst2d, rowsA, rowsB, semA, semB)
            _bar()
            _drain_zero(acc, s2o[p][0], s2o[p][1], rowsA, zb, c, s)
            _bar()

    return pl.kernel(body, out_type=out_type, mesh=_mesh(),
                     scratch_types=scratch, compiler_params=_params())


def _hop_scratch():
    return [
        pltpu.VMEM((CHUNK, 16), jnp.float32),
        pltpu.VMEM((CHUNK, 16), jnp.float32),
        pltpu.VMEM((CHUNK, 16), jnp.float32),
        pltpu.VMEM((CHUNK, 16), jnp.float32),
    ]


def _build_sc2():
    out_type = _q_out() + _o16_out()
    scratch = _base_scratch() + _hop_scratch() + [
        pltpu.VMEM((CHUNK, 16), jnp.float32),
        pltpu.VMEM_SHARED((NPAD, 16), jnp.float32),
    ]

    def body(srcp3, dstp3, z64, z16, c16p, u_qcat,
             oq0, oq1, oq2, oq3, csa, csb,
             src2d, dst2d, srcsh0, srcsh1, rowsA, rowsB, zb, acc, semA, semB,
             csA, csB, cdA, cdB, zb16, acc16):
        c = lax.axis_index("c")
        s = lax.axis_index("s")
        pltpu.sync_copy(z64, zb)
        pltpu.sync_copy(z16, zb16)
        _prolog(srcp3, dstp3, src2d, dst2d, srcsh0, srcsh1, c, s)
        _zero_acc(acc, zb, s)
        _zero_acc(acc16, zb16, s)
        _bar()

        oo = ((oq0, oq2), (oq1, oq3))
        for p, sh in enumerate((srcsh0, srcsh1)):
            _hop_pass(u_qcat, c16p, acc, sh, src2d, dst2d, rowsA, rowsB,
                      csA, csB, cdA, cdB, semA, semB,
                      fold_acc16=acc16 if p == 0 else None, c=c)
            _bar()
            _drain_zero(acc, oo[p][0], oo[p][1], rowsA, zb, c, s)
            if p == 0:
                _drain16_zero(acc16, csa, csb, csA, zb16, c, s)
            _bar()

    return pl.kernel(body, out_type=out_type, mesh=_mesh(),
                     scratch_types=scratch, compiler_params=_params())


def _build_sc3():
    out_type = _q_out()
    scratch = _base_scratch() + _hop_scratch()

    def body(srcp3, dstp3, z64, c16p, u_qcat,
             oq0, oq1, oq2, oq3,
             src2d, dst2d, srcsh0, srcsh1, rowsA, rowsB, zb, acc, semA, semB,
             csA, csB, cdA, cdB):
        c = lax.axis_index("c")
        s = lax.axis_index("s")
        pltpu.sync_copy(z64, zb)
        _prolog(srcp3, dstp3, src2d, dst2d, srcsh0, srcsh1, c, s)
        _zero_acc(acc, zb, s)
        _bar()

        oo = ((oq0, oq2), (oq1, oq3))
        for p, sh in enumerate((srcsh0, srcsh1)):
            _hop_pass(u_qcat, c16p, acc, sh, src2d, dst2d, rowsA, rowsB,
                      csA, csB, cdA, cdB, semA, semB)
            _bar()
            _drain_zero(acc, oo[p][0], oo[p][1], rowsA, zb, c, s)
            _bar()

    return pl.kernel(body, out_type=out_type, mesh=_mesh(),
                     scratch_types=scratch, compiler_params=_params())


_R = 400
_G = N // _R


def _mlp_body(x_ref, w1_ref, b1_ref, w2_ref, b2_ref, w3_ref, b3_ref,
              h_ref, hsq_ref):
    x = x_ref[...]
    h1 = jnp.maximum(
        jnp.dot(x, w1_ref[...], preferred_element_type=jnp.float32)
        + b1_ref[...], 0.0)
    h2 = jnp.maximum(
        jnp.dot(h1, w2_ref[...], preferred_element_type=jnp.float32)
        + b2_ref[...], 0.0)
    h = jnp.dot(h2, w3_ref[...], preferred_element_type=jnp.float32) + b3_ref[...]
    h_ref[...] = h
    hsq_ref[...] = h * h


def _mlp_call(x, w1t, b1r, w2t, b2r, w3t, b3r):
    full = lambda shp: pl.BlockSpec(shp, lambda i: tuple(0 for _ in shp))
    return pl.pallas_call(
        _mlp_body,
        grid=(_G,),
        in_specs=[
            pl.BlockSpec((_R, D), lambda i: (i, 0)),
            full((D, 512)), full((1, 512)),
            full((512, 512)), full((1, 512)),
            full((512, D)), full((1, D)),
        ],
        out_specs=[pl.BlockSpec((_R, D), lambda i: (i, 0))] * 2,
        out_shape=[jax.ShapeDtypeStruct((N, D), jnp.float32)] * 2,
    )(x, w1t, b1r, w2t, b2r, w3t, b3r)


def _stage2_body(s1_ref, s2_ref, dg_ref, h_ref, wf_ref, c16_ref, u1_ref):
    deg = dg_ref[...][:, :1]
    deg_c = jnp.maximum(deg, 1.0)
    s1 = s1_ref[...]
    mean = s1 / deg_c
    var = (s2_ref[...] - 2.0 * mean * s1 + deg * mean * mean) / deg_c
    logit = jnp.sum(var * wf_ref[...], axis=1, keepdims=True)
    cval = jax.nn.sigmoid(logit)
    norm = lax.rsqrt(deg_c)
    c16_ref[...] = jnp.broadcast_to(cval, (_R, 16))
    u1_ref[...] = h_ref[...] * norm


def _stage2_call(s1, s2, dg16, h, wf):
    full = lambda shp: pl.BlockSpec(shp, lambda i: tuple(0 for _ in shp))
    blkD = pl.BlockSpec((_R, D), lambda i: (i, 0))
    blk16 = pl.BlockSpec((_R, 16), lambda i: (i, 0))
    return pl.pallas_call(
        _stage2_body,
        grid=(_G,),
        in_specs=[blkD, blkD, blk16, blkD, full((1, D))],
        out_specs=[blk16, blkD],
        out_shape=[
            jax.ShapeDtypeStruct((N, 16), jnp.float32),
            jax.ShapeDtypeStruct((N, D), jnp.float32),
        ],
    )(s1, s2, dg16, h, wf)


def _hop_combine(o_ref, cs_ref, dg_ref, c16_ref, h_ref):
    deg = dg_ref[...][:, :1]
    deg_c = jnp.maximum(deg, 1.0)
    norm = lax.rsqrt(deg_c)
    cval = c16_ref[...][:, :1]
    c_sum = cs_ref[...][:, :1] + deg * cval
    bb = 1.0 / (2.0 + c_sum / deg_c)
    feat = bb * (o_ref[...] * norm) + 2.0 * bb * h_ref[...]
    return feat, norm


def _stage3_body(o_ref, cs_ref, dg_ref, c16_ref, h_ref, u2_ref):
    feat, norm = _hop_combine(o_ref, cs_ref, dg_ref, c16_ref, h_ref)
    u2_ref[...] = feat * norm


def _stage4_body(o_ref, cs_ref, dg_ref, c16_ref, h_ref, out_ref):
    feat, _ = _hop_combine(o_ref, cs_ref, dg_ref, c16_ref, h_ref)
    m = jnp.max(feat, axis=1, keepdims=True)
    ex = jnp.exp(feat - m)
    out_ref[...] = feat - m - jnp.log(jnp.sum(ex, axis=1, keepdims=True))


def _stage34_call(body, o, cs16, dg16, c16, h):
    blkD = pl.BlockSpec((_R, D), lambda i: (i, 0))
    blk16 = pl.BlockSpec((_R, 16), lambda i: (i, 0))
    return pl.pallas_call(
        body,
        grid=(_G,),
        in_specs=[blkD, blk16, blk16, blk16, blkD],
        out_specs=blkD,
        out_shape=jax.ShapeDtypeStruct((N, D), jnp.float32),
    )(o, cs16, dg16, c16, h)


def kernel(features, edge_index, W1, b1, W2, b2, W3, b3, Wf):
    src = edge_index[0].astype(jnp.int32)
    dst = edge_index[1].astype(jnp.int32)
    pad = E_PAD - E
    srcp = jnp.concatenate([src, jnp.zeros((pad,), jnp.int32)]).reshape(
        NS, NCHUNK, CHUNK)
    dstp = jnp.concatenate([dst, jnp.full((pad,), N, jnp.int32)]).reshape(
        NS, NCHUNK, CHUNK)

    z64 = jnp.zeros((CHUNK, DQ), jnp.float32)
    z16 = jnp.zeros((CHUNK, 16), jnp.float32)
    o16 = jnp.ones((CHUNK, 16), jnp.float32)

    h, hsq = _mlp_call(features, W1.T, b1.reshape(1, -1), W2.T,
                       b2.reshape(1, -1), W3.T, b3.reshape(1, -1))

    qcat = lambda t: jnp.concatenate(
        [t[:, k * DQ:(k + 1) * DQ] for k in range(4)], axis=0)
    uncat = lambda qs: jnp.concatenate([q[:N] for q in qs], axis=1)

    sc1 = _build_sc1()
    (s1q0, s1q1, s1q2, s1q3, s2q0, s2q1, s2q2, s2q3, dga, dgb) = sc1(
        srcp, dstp, z64, o16, z16, qcat(h), qcat(hsq))
    dg16 = (dga + dgb)[:N]
    s1 = uncat((s1q0, s1q1, s1q2, s1q3))
    s2 = uncat((s2q0, s2q1, s2q2, s2q3))

    c16, u1 = _stage2_call(s1, s2, dg16, h, Wf.reshape(1, -1))
    c16p = jnp.concatenate([c16, jnp.zeros((NP16 - N, 16), jnp.float32)])

    sc2 = _build_sc2()
    (oq0, oq1, oq2, oq3, csa, csb) = sc2(srcp, dstp, z64, z16, c16p, qcat(u1))
    cs16 = (csa + csb)[:N]
    o1 = uncat((oq0, oq1, oq2, oq3))

    u2 = _stage34_call(_stage3_body, o1, cs16, dg16, c16, h)

    sc3 = _build_sc3()
    (pq0, pq1, pq2, pq3) = sc3(srcp, dstp, z64, c16p, qcat(u2))
    o2 = uncat((pq0, pq1, pq2, pq3))

    return _stage34_call(_stage4_body, o2, cs16, dg16, c16, h)


# speedup vs baseline: 3.9092x; 1.1551x over previous
import jax
import jax.numpy as jnp
from jax import lax
from jax.experimental import pallas as pl
from jax.experimental.pallas import tpu as pltpu
from jax.experimental.pallas import tpu_sc as plsc

N = 10000
D = 256
DQ = 64
E = 160000

NS = 16
CHUNK = 128
EPT = 10240
NCHUNK = EPT // CHUNK
HCHUNK = NCHUNK // 2
E_PAD = EPT * NS
NPAD = 10240
NP16 = N + 16
RPT = NPAD // NS
RCH = RPT // CHUNK

_mesh = lambda: plsc.VectorSubcoreMesh(core_axis_name="c", subcore_axis_name="s")
_params = lambda: pltpu.CompilerParams(use_tc_tiling_on_sc=False)


def _bar():
    plsc.subcore_barrier()


def _zero_acc(acc, zb, s):
    base = s * RPT
    for k in range(RCH):
        pltpu.sync_copy(zb, acc.at[pl.ds(base + k * CHUNK, CHUNK)])


def _drain_zero(acc, out_a, out_b, bounce, zb, c, s):
    base = s * RPT
    for k in range(RCH):
        sl = pl.ds(base + k * CHUNK, CHUNK)
        pltpu.sync_copy(acc.at[sl], bounce)

        @pl.when(c == 0)
        def _():
            pltpu.sync_copy(bounce, out_a.at[sl])

        @pl.when(c == 1)
        def _():
            pltpu.sync_copy(bounce, out_b.at[sl])

        pltpu.sync_copy(zb, acc.at[sl])


def _prolog(srcp3, dstp3, src2d, dst2d, s):
    pltpu.sync_copy(srcp3.at[s], src2d)
    pltpu.sync_copy(dstp3.at[s], dst2d)


def _plain_pass(tbl, acc, sidx2d, dst2d, rowsA, rowsB, semA, semB,
                fold16=None, c=None):
    pltpu.async_copy(tbl.at[sidx2d.at[0]], rowsA, semA)

    def fold(j):
        if fold16 is not None:
            acc16, ob16 = fold16

            @pl.when(jnp.logical_and(j >= c * HCHUNK, j < (c + 1) * HCHUNK))
            def _():
                pltpu.sync_copy(ob16, acc16.at[dst2d.at[j]], add=True)

    def body(t, _):
        j0 = 2 * t
        pltpu.make_async_copy(tbl.at[sidx2d.at[0]], rowsA, semA).wait()
        pltpu.async_copy(tbl.at[sidx2d.at[j0 + 1]], rowsB, semB)
        fold(j0)
        pltpu.sync_copy(rowsA, acc.at[dst2d.at[j0]], add=True)
        pltpu.make_async_copy(tbl.at[sidx2d.at[0]], rowsB, semB).wait()

        @pl.when(t < HCHUNK - 1)
        def _():
            pltpu.async_copy(tbl.at[sidx2d.at[j0 + 2]], rowsA, semA)

        fold(j0 + 1)
        pltpu.sync_copy(rowsB, acc.at[dst2d.at[j0 + 1]], add=True)
        return 0

    lax.fori_loop(0, HCHUNK, body, 0)


def _hop_issue(tbl, c16p, sidx2d, src2d, dst2d, j, rows, cs, cd, sem):
    pltpu.async_copy(tbl.at[sidx2d.at[j]], rows, sem)
    pltpu.async_copy(c16p.at[src2d.at[j]], cs, sem)
    pltpu.async_copy(c16p.at[dst2d.at[j]], cd, sem)


def _hop_wait(tbl, c16p, sidx2d, src2d, rows, cs, cd, sem):
    pltpu.make_async_copy(tbl.at[sidx2d.at[0]], rows, sem).wait()
    pltpu.make_async_copy(c16p.at[src2d.at[0]], cs, sem).wait()
    pltpu.make_async_copy(c16p.at[src2d.at[0]], cd, sem).wait()


def _scale_rows(rows, cs, cd):
    def rbody(q, _):
        for u in range(4):
            r = 4 * q + u
            ee = cs[r, pl.ds(0, 16)] + cd[r, pl.ds(0, 16)]
            for k in range(DQ // 16):
                rows[r, pl.ds(k * 16, 16)] = rows[r, pl.ds(k * 16, 16)] * ee
        return 0

    lax.fori_loop(0, CHUNK // 4, rbody, 0)


def _hop_pass(tbl, c16p, acc, sidx2d, src2d, dst2d, rowsA, rowsB,
              csA, csB, cdA, cdB, semA, semB, fold_acc16=None, c=None):
    _hop_issue(tbl, c16p, sidx2d, src2d, dst2d, 0, rowsA, csA, cdA, semA)

    def fold(j, cs):
        if fold_acc16 is not None:

            @pl.when(jnp.logical_and(j >= c * HCHUNK, j < (c + 1) * HCHUNK))
            def _():
                pltpu.sync_copy(cs, fold_acc16.at[dst2d.at[j]], add=True)

    def body(t, _):
        j0 = 2 * t
        _hop_wait(tbl, c16p, sidx2d, src2d, rowsA, csA, cdA, semA)
        _hop_issue(tbl, c16p, sidx2d, src2d, dst2d, j0 + 1, rowsB, csB, cdB,
                   semB)
        fold(j0, csA)
        _scale_rows(rowsA, csA, cdA)
        pltpu.sync_copy(rowsA, acc.at[dst2d.at[j0]], add=True)
        _hop_wait(tbl, c16p, sidx2d, src2d, rowsB, csB, cdB, semB)

        @pl.when(t < HCHUNK - 1)
        def _():
            _hop_issue(tbl, c16p, sidx2d, src2d, dst2d, j0 + 2, rowsA, csA,
                       cdA, semA)

        fold(j0 + 1, csB)
        _scale_rows(rowsB, csB, cdB)
        pltpu.sync_copy(rowsB, acc.at[dst2d.at[j0 + 1]], add=True)
        return 0

    lax.fori_loop(0, HCHUNK, body, 0)


def _drain16_zero(acc16, out_a, out_b, bounce16, zb16, c, s):
    base = s * RPT
    for k in range(RCH):
        sl = pl.ds(base + k * CHUNK, CHUNK)
        pltpu.sync_copy(acc16.at[sl], bounce16)

        @pl.when(c == 0)
        def _():
            pltpu.sync_copy(bounce16, out_a.at[sl])

        @pl.when(c == 1)
        def _():
            pltpu.sync_copy(bounce16, out_b.at[sl])

        pltpu.sync_copy(zb16, acc16.at[sl])


def _q_out():
    return tuple(jax.ShapeDtypeStruct((NPAD, DQ), jnp.float32) for _ in range(4))


def _o16_out():
    return (jax.ShapeDtypeStruct((NPAD, 16), jnp.float32),
            jax.ShapeDtypeStruct((NPAD, 16), jnp.float32))


def _base_scratch():
    return [
        pltpu.VMEM((NCHUNK, CHUNK), jnp.int32),
        pltpu.VMEM((NCHUNK, CHUNK), jnp.int32),
        pltpu.VMEM((CHUNK, DQ), jnp.float32),
        pltpu.VMEM((CHUNK, DQ), jnp.float32),
        pltpu.VMEM((CHUNK, DQ), jnp.float32),
        pltpu.VMEM_SHARED((NPAD, DQ), jnp.float32),
        pltpu.SemaphoreType.DMA,
        pltpu.SemaphoreType.DMA,
    ]


def _per_core(c, fn_a, fn_b):
    @pl.when(c == 0)
    def _():
        fn_a()

    @pl.when(c == 1)
    def _():
        fn_b()


def _build_sc1():
    out_type = _q_out() + _q_out() + _o16_out()
    scratch = _base_scratch() + [
        pltpu.VMEM((CHUNK, 16), jnp.float32),
        pltpu.VMEM((CHUNK, 16), jnp.float32),
        pltpu.VMEM((CHUNK, 16), jnp.float32),
        pltpu.VMEM_SHARED((NPAD, 16), jnp.float32),
    ]

    def body(srcp3, dstp3, z64, o16, z16, hq0, hq1, hq2, hq3,
             h2q0, h2q1, h2q2, h2q3,
             s1q0, s1q1, s1q2, s1q3, s2q0, s2q1, s2q2, s2q3, dga, dgb,
             src2d, dst2d, rowsA, rowsB, zb, acc, semA, semB,
             rows16, zb16, ob16, acc16):
        c = lax.axis_index("c")
        s = lax.axis_index("s")
        pltpu.sync_copy(z64, zb)
        pltpu.sync_copy(z16, zb16)
        pltpu.sync_copy(o16, ob16)
        _prolog(srcp3, dstp3, src2d, dst2d, s)
        _zero_acc(acc, zb, s)
        _zero_acc(acc16, zb16, s)
        _bar()

        htb = (hq0, hq1, hq2, hq3)
        h2tb = (h2q0, h2q1, h2q2, h2q3)
        s1o = ((s1q0, s1q2), (s1q1, s1q3))
        s2o = ((s2q0, s2q2), (s2q1, s2q3))
        for p in range(2):
            fold = (acc16, ob16) if p == 0 else None
            _per_core(
                c,
                lambda p=p: _plain_pass(htb[p], acc, src2d, dst2d, rowsA,
                                        rowsB, semA, semB, fold16=fold, c=c),
                lambda p=p: _plain_pass(htb[2 + p], acc, src2d, dst2d, rowsA,
                                        rowsB, semA, semB, fold16=fold, c=c))
            _bar()
            _drain_zero(acc, s1o[p][0], s1o[p][1], rowsA, zb, c, s)
            if p == 0:
                _drain16_zero(acc16, dga, dgb, rows16, zb16, c, s)
            _bar()
        for p in range(2):
            _per_core(
                c,
                lambda p=p: _plain_pass(h2tb[p], acc, src2d, dst2d, rowsA,
                                        rowsB, semA, semB),
                lambda p=p: _plain_pass(h2tb[2 + p], acc, src2d, dst2d, rowsA,
                                        rowsB, semA, semB))
            _bar()
            _drain_zero(acc, s2o[p][0], s2o[p][1], rowsA, zb, c, s)
            _bar()

    return pl.kernel(body, out_type=out_type, mesh=_mesh(),
                     scratch_types=scratch, compiler_params=_params())


def _hop_scratch():
    return [
        pltpu.VMEM((CHUNK, 16), jnp.float32),
        pltpu.VMEM((CHUNK, 16), jnp.float32),
        pltpu.VMEM((CHUNK, 16), jnp.float32),
        pltpu.VMEM((CHUNK, 16), jnp.float32),
    ]


def _build_sc2():
    out_type = _q_out() + _o16_out()
    scratch = _base_scratch() + _hop_scratch() + [
        pltpu.VMEM((CHUNK, 16), jnp.float32),
        pltpu.VMEM_SHARED((NPAD, 16), jnp.float32),
    ]

    def body(srcp3, dstp3, z64, z16, c16p, uq0, uq1, uq2, uq3,
             oq0, oq1, oq2, oq3, csa, csb,
             src2d, dst2d, rowsA, rowsB, zb, acc, semA, semB,
             csA, csB, cdA, cdB, zb16, acc16):
        c = lax.axis_index("c")
        s = lax.axis_index("s")
        pltpu.sync_copy(z64, zb)
        pltpu.sync_copy(z16, zb16)
        _prolog(srcp3, dstp3, src2d, dst2d, s)
        _zero_acc(acc, zb, s)
        _zero_acc(acc16, zb16, s)
        _bar()

        utb = (uq0, uq1, uq2, uq3)
        oo = ((oq0, oq2), (oq1, oq3))
        for p in range(2):
            fold = acc16 if p == 0 else None
            _per_core(
                c,
                lambda p=p: _hop_pass(utb[p], c16p, acc, src2d, src2d, dst2d,
                                      rowsA, rowsB, csA, csB, cdA, cdB,
                                      semA, semB, fold_acc16=fold, c=c),
                lambda p=p: _hop_pass(utb[2 + p], c16p, acc, src2d, src2d,
                                      dst2d, rowsA, rowsB, csA, csB, cdA, cdB,
                                      semA, semB, fold_acc16=fold, c=c))
            _bar()
            _drain_zero(acc, oo[p][0], oo[p][1], rowsA, zb, c, s)
            if p == 0:
                _drain16_zero(acc16, csa, csb, csA, zb16, c, s)
            _bar()

    return pl.kernel(body, out_type=out_type, mesh=_mesh(),
                     scratch_types=scratch, compiler_params=_params())


def _build_sc3():
    out_type = _q_out()
    scratch = _base_scratch() + _hop_scratch()

    def body(srcp3, dstp3, z64, c16p, uq0, uq1, uq2, uq3,
             oq0, oq1, oq2, oq3,
             src2d, dst2d, rowsA, rowsB, zb, acc, semA, semB,
             csA, csB, cdA, cdB):
        c = lax.axis_index("c")
        s = lax.axis_index("s")
        pltpu.sync_copy(z64, zb)
        _prolog(srcp3, dstp3, src2d, dst2d, s)
        _zero_acc(acc, zb, s)
        _bar()

        utb = (uq0, uq1, uq2, uq3)
        oo = ((oq0, oq2), (oq1, oq3))
        for p in range(2):
            _per_core(
                c,
                lambda p=p: _hop_pass(utb[p], c16p, acc, src2d, src2d, dst2d,
                                      rowsA, rowsB, csA, csB, cdA, cdB,
                                      semA, semB),
                lambda p=p: _hop_pass(utb[2 + p], c16p, acc, src2d, src2d,
                                      dst2d, rowsA, rowsB, csA, csB, cdA, cdB,
                                      semA, semB))
            _bar()
            _drain_zero(acc, oo[p][0], oo[p][1], rowsA, zb, c, s)
            _bar()

    return pl.kernel(body, out_type=out_type, mesh=_mesh(),
                     scratch_types=scratch, compiler_params=_params())


_R = 400
_G = N // _R

_blkD = lambda: pl.BlockSpec((_R, D), lambda i: (i, 0))
_blkQ = lambda: pl.BlockSpec((_R, DQ), lambda i: (i, 0))
_blk16 = lambda: pl.BlockSpec((_R, 16), lambda i: (i, 0))
_full = lambda shp: pl.BlockSpec(shp, lambda i: tuple(0 for _ in shp))
_outQ = lambda: [jax.ShapeDtypeStruct((N, DQ), jnp.float32)] * 4


def _wrq(refs, x):
    for k in range(4):
        refs[k][...] = x[:, k * DQ:(k + 1) * DQ]


def _rdq(refs):
    return jnp.concatenate([r[...] for r in refs], axis=1)


def _mlp_body(x_ref, w1_ref, b1_ref, w2_ref, b2_ref, w3_ref, b3_ref,
              h_ref, *q_refs):
    x = x_ref[...]
    h1 = jnp.maximum(
        jnp.dot(x, w1_ref[...], preferred_element_type=jnp.float32)
        + b1_ref[...], 0.0)
    h2 = jnp.maximum(
        jnp.dot(h1, w2_ref[...], preferred_element_type=jnp.float32)
        + b2_ref[...], 0.0)
    h = jnp.dot(h2, w3_ref[...], preferred_element_type=jnp.float32) + b3_ref[...]
    h_ref[...] = h
    _wrq(q_refs[:4], h)
    _wrq(q_refs[4:], h * h)


def _mlp_call(x, w1t, b1r, w2t, b2r, w3t, b3r):
    return pl.pallas_call(
        _mlp_body,
        grid=(_G,),
        in_specs=[
            _blkD(),
            _full((D, 512)), _full((1, 512)),
            _full((512, 512)), _full((1, 512)),
            _full((512, D)), _full((1, D)),
        ],
        out_specs=[_blkD()] + [_blkQ()] * 8,
        out_shape=[jax.ShapeDtypeStruct((N, D), jnp.float32)] + _outQ() + _outQ(),
    )(x, w1t, b1r, w2t, b2r, w3t, b3r)


def _stage2_body(s1q0, s1q1, s1q2, s1q3, s2q0, s2q1, s2q2, s2q3,
                 dga_ref, dgb_ref, h_ref, wf_ref, c16_ref, *u1_refs):
    deg = (dga_ref[...] + dgb_ref[...])[:, :1]
    deg_c = jnp.maximum(deg, 1.0)
    s1 = _rdq((s1q0, s1q1, s1q2, s1q3))
    s2 = _rdq((s2q0, s2q1, s2q2, s2q3))
    mean = s1 / deg_c
    var = (s2 - 2.0 * mean * s1 + deg * mean * mean) / deg_c
    logit = jnp.sum(var * wf_ref[...], axis=1, keepdims=True)
    cval = jax.nn.sigmoid(logit)
    norm = lax.rsqrt(deg_c)
    c16_ref[...] = jnp.broadcast_to(cval, (_R, 16))
    _wrq(u1_refs, h_ref[...] * norm)


def _stage2_call(s1q, s2q, dga, dgb, h, wf):
    return pl.pallas_call(
        _stage2_body,
        grid=(_G,),
        in_specs=[_blkQ()] * 8 + [_blk16(), _blk16(), _blkD(), _full((1, D))],
        out_specs=[_blk16()] + [_blkQ()] * 4,
        out_shape=[jax.ShapeDtypeStruct((N, 16), jnp.float32)] + _outQ(),
    )(*s1q, *s2q, dga, dgb, h, wf)


def _hop_combine(oq, csa_ref, csb_ref, dga_ref, dgb_ref, c16_ref, h_ref):
    deg = (dga_ref[...] + dgb_ref[...])[:, :1]
    deg_c = jnp.maximum(deg, 1.0)
    norm = lax.rsqrt(deg_c)
    cval = c16_ref[...][:, :1]
    c_sum = (csa_ref[...] + csb_ref[...])[:, :1] + deg * cval
    bb = 1.0 / (2.0 + c_sum / deg_c)
    feat = bb * (_rdq(oq) * norm) + 2.0 * bb * h_ref[...]
    return feat, norm


def _stage3_body(oq0, oq1, oq2, oq3, csa_ref, csb_ref, dga_ref, dgb_ref,
                 c16_ref, h_ref, *u2_refs):
    feat, norm = _hop_combine((oq0, oq1, oq2, oq3), csa_ref, csb_ref,
                              dga_ref, dgb_ref, c16_ref, h_ref)
    _wrq(u2_refs, feat * norm)


def _stage4_body(oq0, oq1, oq2, oq3, csa_ref, csb_ref, dga_ref, dgb_ref,
                 c16_ref, h_ref, out_ref):
    feat, _ = _hop_combine((oq0, oq1, oq2, oq3), csa_ref, csb_ref,
                           dga_ref, dgb_ref, c16_ref, h_ref)
    m = jnp.max(feat, axis=1, keepdims=True)
    ex = jnp.exp(feat - m)
    out_ref[...] = feat - m - jnp.log(jnp.sum(ex, axis=1, keepdims=True))


def _stage3_call(oq, csa, csb, dga, dgb, c16, h):
    return pl.pallas_call(
        _stage3_body,
        grid=(_G,),
        in_specs=[_blkQ()] * 4 + [_blk16()] * 5 + [_blkD()],
        out_specs=[_blkQ()] * 4,
        out_shape=_outQ(),
    )(*oq, csa, csb, dga, dgb, c16, h)


def _stage4_call(oq, csa, csb, dga, dgb, c16, h):
    return pl.pallas_call(
        _stage4_body,
        grid=(_G,),
        in_specs=[_blkQ()] * 4 + [_blk16()] * 5 + [_blkD()],
        out_specs=_blkD(),
        out_shape=jax.ShapeDtypeStruct((N, D), jnp.float32),
    )(*oq, csa, csb, dga, dgb, c16, h)


def kernel(features, edge_index, W1, b1, W2, b2, W3, b3, Wf):
    src = edge_index[0].astype(jnp.int32)
    dst = edge_index[1].astype(jnp.int32)
    pad = E_PAD - E
    srcp = jnp.concatenate([src, jnp.zeros((pad,), jnp.int32)]).reshape(
        NS, NCHUNK, CHUNK)
    dstp = jnp.concatenate([dst, jnp.full((pad,), N, jnp.int32)]).reshape(
        NS, NCHUNK, CHUNK)

    z64 = jnp.zeros((CHUNK, DQ), jnp.float32)
    z16 = jnp.zeros((CHUNK, 16), jnp.float32)
    o16 = jnp.ones((CHUNK, 16), jnp.float32)

    h, *hqs = _mlp_call(features, W1.T, b1.reshape(1, -1), W2.T,
                        b2.reshape(1, -1), W3.T, b3.reshape(1, -1))
    hq, h2q = hqs[:4], hqs[4:]

    sc1 = _build_sc1()
    (s1q0, s1q1, s1q2, s1q3, s2q0, s2q1, s2q2, s2q3, dga, dgb) = sc1(
        srcp, dstp, z64, o16, z16, *hq, *h2q)

    c16, *u1q = _stage2_call((s1q0, s1q1, s1q2, s1q3),
                             (s2q0, s2q1, s2q2, s2q3), dga, dgb, h,
                             Wf.reshape(1, -1))
    c16p = jnp.concatenate([c16, jnp.zeros((NP16 - N, 16), jnp.float32)])

    sc2 = _build_sc2()
    (oq0, oq1, oq2, oq3, csa, csb) = sc2(srcp, dstp, z64, z16, c16p, *u1q)

    u2q = _stage3_call((oq0, oq1, oq2, oq3), csa, csb, dga, dgb, c16, h)

    sc3 = _build_sc3()
    pq = sc3(srcp, dstp, z64, c16p, *u2q)

    return _stage4_call(pq, csa, csb, dga, dgb, c16, h)


# speedup vs baseline: 4.4762x; 1.1450x over previous
import jax
import jax.numpy as jnp
from jax import lax
from jax.experimental import pallas as pl
from jax.experimental.pallas import tpu as pltpu
from jax.experimental.pallas import tpu_sc as plsc

N = 10000
D = 256
DQ = 64
E = 160000

NS = 16
CHUNK = 128
EPT = 10240
NCHUNK = EPT // CHUNK
HCHUNK = NCHUNK // 2
E_PAD = EPT * NS
NPAD = 10240
NP16 = N + 16
RPT = NPAD // NS
RCH = RPT // CHUNK

_mesh = lambda: plsc.VectorSubcoreMesh(core_axis_name="c", subcore_axis_name="s")
_params = lambda: pltpu.CompilerParams(use_tc_tiling_on_sc=False)


def _bar():
    plsc.subcore_barrier()


def _zero_acc(acc, zb, s):
    base = s * RPT
    for k in range(RCH):
        pltpu.sync_copy(zb, acc.at[pl.ds(base + k * CHUNK, CHUNK)])


def _drain_zero(acc, out_a, out_b, bounce, zb, c, s):
    base = s * RPT
    for k in range(RCH):
        sl = pl.ds(base + k * CHUNK, CHUNK)
        pltpu.sync_copy(acc.at[sl], bounce)

        @pl.when(c == 0)
        def _():
            pltpu.sync_copy(bounce, out_a.at[sl])

        @pl.when(c == 1)
        def _():
            pltpu.sync_copy(bounce, out_b.at[sl])

        pltpu.sync_copy(zb, acc.at[sl])


def _prolog(srcp3, dstp3, src2d, dst2d, s):
    pltpu.sync_copy(srcp3.at[s], src2d)
    pltpu.sync_copy(dstp3.at[s], dst2d)


def _plain_pass(tbl, acc, sidx2d, dst2d, rows4, gsems, ssems,
                fold16=None, c=None):
    QT = NCHUNK // 4

    def fold(j):
        if fold16 is not None:
            acc16, ob16 = fold16

            @pl.when(jnp.logical_and(j >= c * HCHUNK, j < (c + 1) * HCHUNK))
            def _():
                pltpu.sync_copy(ob16, acc16.at[dst2d.at[j]], add=True)

    for x in range(3):
        pltpu.async_copy(tbl.at[sidx2d.at[x]], rows4[x], gsems[x])

    def body(t, _):
        for u in range(4):
            j = 4 * t + u
            pltpu.make_async_copy(tbl.at[sidx2d.at[0]], rows4[u],
                                  gsems[u]).wait()
            fold(j)
            pltpu.async_copy(rows4[u], acc.at[dst2d.at[j]], ssems[u],
                             add=True)
            y = (u + 3) % 4
            if u == 0:
                @pl.when(t > 0)
                def _():
                    pltpu.make_async_copy(rows4[y], acc.at[dst2d.at[0]],
                                          ssems[y]).wait()
                pltpu.async_copy(tbl.at[sidx2d.at[j + 3]], rows4[y], gsems[y])
            else:
                @pl.when(t < QT - 1)
                def _():
                    pltpu.make_async_copy(rows4[y], acc.at[dst2d.at[0]],
                                          ssems[y]).wait()
                    pltpu.async_copy(tbl.at[sidx2d.at[j + 3]], rows4[y],
                                     gsems[y])
        return 0

    lax.fori_loop(0, QT, body, 0)
    for x in range(4):
        pltpu.make_async_copy(rows4[x], acc.at[dst2d.at[0]], ssems[x]).wait()


def _scale_rows(rows, cs, cd):
    def rbody(q, _):
        for u in range(4):
            r = 4 * q + u
            ee = cs[r, pl.ds(0, 16)] + cd[r, pl.ds(0, 16)]
            for k in range(DQ // 16):
                rows[r, pl.ds(k * 16, 16)] = rows[r, pl.ds(k * 16, 16)] * ee
        return 0

    lax.fori_loop(0, CHUNK // 4, rbody, 0)


def _hop_pass(tbl, c16p, acc, sidx2d, src2d, dst2d, rows4, cs4, cd4,
              gsems, ssems, fold_acc16=None, c=None):
    QT = NCHUNK // 4

    def issue(j, x):
        pltpu.async_copy(tbl.at[sidx2d.at[j]], rows4[x], gsems[x])
        pltpu.async_copy(c16p.at[src2d.at[j]], cs4[x], gsems[x])
        pltpu.async_copy(c16p.at[dst2d.at[j]], cd4[x], gsems[x])

    def gwait(x):
        pltpu.make_async_copy(tbl.at[sidx2d.at[0]], rows4[x], gsems[x]).wait()
        pltpu.make_async_copy(c16p.at[src2d.at[0]], cs4[x], gsems[x]).wait()
        pltpu.make_async_copy(c16p.at[src2d.at[0]], cd4[x], gsems[x]).wait()

    def fold(j, x):
        if fold_acc16 is not None:

            @pl.when(jnp.logical_and(j >= c * HCHUNK, j < (c + 1) * HCHUNK))
            def _():
                pltpu.sync_copy(cs4[x], fold_acc16.at[dst2d.at[j]], add=True)

    for x in range(3):
        issue(x, x)

    def body(t, _):
        for u in range(4):
            j = 4 * t + u
            gwait(u)
            fold(j, u)
            _scale_rows(rows4[u], cs4[u], cd4[u])
            pltpu.async_copy(rows4[u], acc.at[dst2d.at[j]], ssems[u],
                             add=True)
            y = (u + 3) % 4
            if u == 0:
                @pl.when(t > 0)
                def _():
                    pltpu.make_async_copy(rows4[y], acc.at[dst2d.at[0]],
                                          ssems[y]).wait()
                issue(j + 3, y)
            else:
                @pl.when(t < QT - 1)
                def _():
                    pltpu.make_async_copy(rows4[y], acc.at[dst2d.at[0]],
                                          ssems[y]).wait()
                    issue(j + 3, y)
        return 0

    lax.fori_loop(0, QT, body, 0)
    for x in range(4):
        pltpu.make_async_copy(rows4[x], acc.at[dst2d.at[0]], ssems[x]).wait()


def _drain16_zero(acc16, out_a, out_b, bounce16, zb16, c, s):
    base = s * RPT
    for k in range(RCH):
        sl = pl.ds(base + k * CHUNK, CHUNK)
        pltpu.sync_copy(acc16.at[sl], bounce16)

        @pl.when(c == 0)
        def _():
            pltpu.sync_copy(bounce16, out_a.at[sl])

        @pl.when(c == 1)
        def _():
            pltpu.sync_copy(bounce16, out_b.at[sl])

        pltpu.sync_copy(zb16, acc16.at[sl])


def _q_out():
    return tuple(jax.ShapeDtypeStruct((NPAD, DQ), jnp.float32) for _ in range(4))


def _o16_out():
    return (jax.ShapeDtypeStruct((NPAD, 16), jnp.float32),
            jax.ShapeDtypeStruct((NPAD, 16), jnp.float32))


def _base_scratch():
    return [
        pltpu.VMEM((NCHUNK, CHUNK), jnp.int32),
        pltpu.VMEM((NCHUNK, CHUNK), jnp.int32),
        pltpu.VMEM((CHUNK, DQ), jnp.float32),
        pltpu.VMEM((CHUNK, DQ), jnp.float32),
        pltpu.VMEM((CHUNK, DQ), jnp.float32),
        pltpu.VMEM((CHUNK, DQ), jnp.float32),
        pltpu.VMEM((CHUNK, DQ), jnp.float32),
        pltpu.VMEM_SHARED((NPAD, DQ), jnp.float32),
        pltpu.SemaphoreType.DMA,
        pltpu.SemaphoreType.DMA,
        pltpu.SemaphoreType.DMA,
        pltpu.SemaphoreType.DMA,
        pltpu.SemaphoreType.DMA,
        pltpu.SemaphoreType.DMA,
        pltpu.SemaphoreType.DMA,
        pltpu.SemaphoreType.DMA,
    ]


def _per_core(c, fn_a, fn_b):
    @pl.when(c == 0)
    def _():
        fn_a()

    @pl.when(c == 1)
    def _():
        fn_b()


def _build_sc1():
    out_type = _q_out() + _q_out() + _o16_out()
    scratch = _base_scratch() + [
        pltpu.VMEM((CHUNK, 16), jnp.float32),
        pltpu.VMEM((CHUNK, 16), jnp.float32),
        pltpu.VMEM((CHUNK, 16), jnp.float32),
        pltpu.VMEM_SHARED((NPAD, 16), jnp.float32),
    ]

    def body(srcp3, dstp3, z64, o16, z16, hq0, hq1, hq2, hq3,
             h2q0, h2q1, h2q2, h2q3,
             s1q0, s1q1, s1q2, s1q3, s2q0, s2q1, s2q2, s2q3, dga, dgb,
             src2d, dst2d, r0, r1, r2, r3, zb, acc,
             g0, g1, g2, g3, ss0, ss1, ss2, ss3,
             rows16, zb16, ob16, acc16):
        rows4 = (r0, r1, r2, r3)
        gsems = (g0, g1, g2, g3)
        ssems = (ss0, ss1, ss2, ss3)
        c = lax.axis_index("c")
        s = lax.axis_index("s")
        pltpu.sync_copy(z64, zb)
        pltpu.sync_copy(z16, zb16)
        pltpu.sync_copy(o16, ob16)
        _prolog(srcp3, dstp3, src2d, dst2d, s)
        _zero_acc(acc, zb, s)
        _zero_acc(acc16, zb16, s)
        _bar()

        htb = (hq0, hq1, hq2, hq3)
        h2tb = (h2q0, h2q1, h2q2, h2q3)
        s1o = ((s1q0, s1q2), (s1q1, s1q3))
        s2o = ((s2q0, s2q2), (s2q1, s2q3))
        for p in range(2):
            fold = (acc16, ob16) if p == 0 else None
            _per_core(
                c,
                lambda p=p: _plain_pass(htb[p], acc, src2d, dst2d, rows4,
                                        gsems, ssems, fold16=fold, c=c),
                lambda p=p: _plain_pass(htb[2 + p], acc, src2d, dst2d, rows4,
                                        gsems, ssems, fold16=fold, c=c))
            _bar()
            _drain_zero(acc, s1o[p][0], s1o[p][1], r0, zb, c, s)
            if p == 0:
                _drain16_zero(acc16, dga, dgb, rows16, zb16, c, s)
            _bar()
        for p in range(2):
            _per_core(
                c,
                lambda p=p: _plain_pass(h2tb[p], acc, src2d, dst2d, rows4,
                                        gsems, ssems),
                lambda p=p: _plain_pass(h2tb[2 + p], acc, src2d, dst2d, rows4,
                                        gsems, ssems))
            _bar()
            _drain_zero(acc, s2o[p][0], s2o[p][1], r0, zb, c, s)
            _bar()

    return pl.kernel(body, out_type=out_type, mesh=_mesh(),
                     scratch_types=scratch, compiler_params=_params())


def _hop_scratch():
    return [pltpu.VMEM((CHUNK, 16), jnp.float32) for _ in range(8)]


def _build_sc2():
    out_type = _q_out() + _o16_out()
    scratch = _base_scratch() + _hop_scratch() + [
        pltpu.VMEM((CHUNK, 16), jnp.float32),
        pltpu.VMEM_SHARED((NPAD, 16), jnp.float32),
    ]

    def body(srcp3, dstp3, z64, z16, c16p, uq0, uq1, uq2, uq3,
             oq0, oq1, oq2, oq3, csa, csb,
             src2d, dst2d, r0, r1, r2, r3, zb, acc,
             g0, g1, g2, g3, ss0, ss1, ss2, ss3,
             c0, c1, c2, c3, d0, d1, d2, d3, zb16, acc16):
        rows4 = (r0, r1, r2, r3)
        gsems = (g0, g1, g2, g3)
        ssems = (ss0, ss1, ss2, ss3)
        cs4 = (c0, c1, c2, c3)
        cd4 = (d0, d1, d2, d3)
        c = lax.axis_index("c")
        s = lax.axis_index("s")
        pltpu.sync_copy(z64, zb)
        pltpu.sync_copy(z16, zb16)
        _prolog(srcp3, dstp3, src2d, dst2d, s)
        _zero_acc(acc, zb, s)
        _zero_acc(acc16, zb16, s)
        _bar()

        utb = (uq0, uq1, uq2, uq3)
        oo = ((oq0, oq2), (oq1, oq3))
        for p in range(2):
            fold = acc16 if p == 0 else None
            _per_core(
                c,
                lambda p=p: _hop_pass(utb[p], c16p, acc, src2d, src2d, dst2d,
                                      rows4, cs4, cd4, gsems, ssems,
                                      fold_acc16=fold, c=c),
                lambda p=p: _hop_pass(utb[2 + p], c16p, acc, src2d, src2d,
                                      dst2d, rows4, cs4, cd4, gsems, ssems,
                                      fold_acc16=fold, c=c))
            _bar()
            _drain_zero(acc, oo[p][0], oo[p][1], r0, zb, c, s)
            if p == 0:
                _drain16_zero(acc16, csa, csb, c0, zb16, c, s)
            _bar()

    return pl.kernel(body, out_type=out_type, mesh=_mesh(),
                     scratch_types=scratch, compiler_params=_params())


def _build_sc3():
    out_type = _q_out()
    scratch = _base_scratch() + _hop_scratch()

    def body(srcp3, dstp3, z64, c16p, uq0, uq1, uq2, uq3,
             oq0, oq1, oq2, oq3,
             src2d, dst2d, r0, r1, r2, r3, zb, acc,
             g0, g1, g2, g3, ss0, ss1, ss2, ss3,
             c0, c1, c2, c3, d0, d1, d2, d3):
        rows4 = (r0, r1, r2, r3)
        gsems = (g0, g1, g2, g3)
        ssems = (ss0, ss1, ss2, ss3)
        cs4 = (c0, c1, c2, c3)
        cd4 = (d0, d1, d2, d3)
        c = lax.axis_index("c")
        s = lax.axis_index("s")
        pltpu.sync_copy(z64, zb)
        _prolog(srcp3, dstp3, src2d, dst2d, s)
        _zero_acc(acc, zb, s)
        _bar()

        utb = (uq0, uq1, uq2, uq3)
        oo = ((oq0, oq2), (oq1, oq3))
        for p in range(2):
            _per_core(
                c,
                lambda p=p: _hop_pass(utb[p], c16p, acc, src2d, src2d, dst2d,
                                      rows4, cs4, cd4, gsems, ssems),
                lambda p=p: _hop_pass(utb[2 + p], c16p, acc, src2d, src2d,
                                      dst2d, rows4, cs4, cd4, gsems, ssems))
            _bar()
            _drain_zero(acc, oo[p][0], oo[p][1], r0, zb, c, s)
            _bar()

    return pl.kernel(body, out_type=out_type, mesh=_mesh(),
                     scratch_types=scratch, compiler_params=_params())


_R = 400
_G = N // _R

_blkD = lambda: pl.BlockSpec((_R, D), lambda i: (i, 0))
_blkQ = lambda: pl.BlockSpec((_R, DQ), lambda i: (i, 0))
_blk16 = lambda: pl.BlockSpec((_R, 16), lambda i: (i, 0))
_full = lambda shp: pl.BlockSpec(shp, lambda i: tuple(0 for _ in shp))
_outQ = lambda: [jax.ShapeDtypeStruct((N, DQ), jnp.float32)] * 4


def _wrq(refs, x):
    for k in range(4):
        refs[k][...] = x[:, k * DQ:(k + 1) * DQ]


def _rdq(refs):
    return jnp.concatenate([r[...] for r in refs], axis=1)


def _mlp_body(x_ref, w1_ref, b1_ref, w2_ref, b2_ref, w3_ref, b3_ref,
              h_ref, *q_refs):
    x = x_ref[...]
    h1 = jnp.maximum(
        jnp.dot(x, w1_ref[...], preferred_element_type=jnp.float32)
        + b1_ref[...], 0.0)
    h2 = jnp.maximum(
        jnp.dot(h1, w2_ref[...], preferred_element_type=jnp.float32)
        + b2_ref[...], 0.0)
    h = jnp.dot(h2, w3_ref[...], preferred_element_type=jnp.float32) + b3_ref[...]
    h_ref[...] = h
    _wrq(q_refs[:4], h)
    _wrq(q_refs[4:], h * h)


def _mlp_call(x, w1t, b1r, w2t, b2r, w3t, b3r):
    return pl.pallas_call(
        _mlp_body,
        grid=(_G,),
        in_specs=[
            _blkD(),
            _full((D, 512)), _full((1, 512)),
            _full((512, 512)), _full((1, 512)),
            _full((512, D)), _full((1, D)),
        ],
        out_specs=[_blkD()] + [_blkQ()] * 8,
        out_shape=[jax.ShapeDtypeStruct((N, D), jnp.float32)] + _outQ() + _outQ(),
    )(x, w1t, b1r, w2t, b2r, w3t, b3r)


def _stage2_body(s1q0, s1q1, s1q2, s1q3, s2q0, s2q1, s2q2, s2q3,
                 dga_ref, dgb_ref, h_ref, wf_ref, c16_ref, *u1_refs):
    deg = (dga_ref[...] + dgb_ref[...])[:, :1]
    deg_c = jnp.maximum(deg, 1.0)
    s1 = _rdq((s1q0, s1q1, s1q2, s1q3))
    s2 = _rdq((s2q0, s2q1, s2q2, s2q3))
    mean = s1 / deg_c
    var = (s2 - 2.0 * mean * s1 + deg * mean * mean) / deg_c
    logit = jnp.sum(var * wf_ref[...], axis=1, keepdims=True)
    cval = jax.nn.sigmoid(logit)
    norm = lax.rsqrt(deg_c)
    c16_ref[...] = jnp.broadcast_to(cval, (_R, 16))
    _wrq(u1_refs, h_ref[...] * norm)


def _stage2_call(s1q, s2q, dga, dgb, h, wf):
    return pl.pallas_call(
        _stage2_body,
        grid=(_G,),
        in_specs=[_blkQ()] * 8 + [_blk16(), _blk16(), _blkD(), _full((1, D))],
        out_specs=[_blk16()] + [_blkQ()] * 4,
        out_shape=[jax.ShapeDtypeStruct((N, 16), jnp.float32)] + _outQ(),
    )(*s1q, *s2q, dga, dgb, h, wf)


def _hop_combine(oq, csa_ref, csb_ref, dga_ref, dgb_ref, c16_ref, h_ref):
    deg = (dga_ref[...] + dgb_ref[...])[:, :1]
    deg_c = jnp.maximum(deg, 1.0)
    norm = lax.rsqrt(deg_c)
    cval = c16_ref[...][:, :1]
    c_sum = (csa_ref[...] + csb_ref[...])[:, :1] + deg * cval
    bb = 1.0 / (2.0 + c_sum / deg_c)
    feat = bb * (_rdq(oq) * norm) + 2.0 * bb * h_ref[...]
    return feat, norm


def _stage3_body(oq0, oq1, oq2, oq3, csa_ref, csb_ref, dga_ref, dgb_ref,
                 c16_ref, h_ref, *u2_refs):
    feat, norm = _hop_combine((oq0, oq1, oq2, oq3), csa_ref, csb_ref,
                              dga_ref, dgb_ref, c16_ref, h_ref)
    _wrq(u2_refs, feat * norm)


def _stage4_body(oq0, oq1, oq2, oq3, csa_ref, csb_ref, dga_ref, dgb_ref,
                 c16_ref, h_ref, out_ref):
    feat, _ = _hop_combine((oq0, oq1, oq2, oq3), csa_ref, csb_ref,
                           dga_ref, dgb_ref, c16_ref, h_ref)
    m = jnp.max(feat, axis=1, keepdims=True)
    ex = jnp.exp(feat - m)
    out_ref[...] = feat - m - jnp.log(jnp.sum(ex, axis=1, keepdims=True))


def _stage3_call(oq, csa, csb, dga, dgb, c16, h):
    return pl.pallas_call(
        _stage3_body,
        grid=(_G,),
        in_specs=[_blkQ()] * 4 + [_blk16()] * 5 + [_blkD()],
        out_specs=[_blkQ()] * 4,
        out_shape=_outQ(),
    )(*oq, csa, csb, dga, dgb, c16, h)


def _stage4_call(oq, csa, csb, dga, dgb, c16, h):
    return pl.pallas_call(
        _stage4_body,
        grid=(_G,),
        in_specs=[_blkQ()] * 4 + [_blk16()] * 5 + [_blkD()],
        out_specs=_blkD(),
        out_shape=jax.ShapeDtypeStruct((N, D), jnp.float32),
    )(*oq, csa, csb, dga, dgb, c16, h)


def kernel(features, edge_index, W1, b1, W2, b2, W3, b3, Wf):
    src = edge_index[0].astype(jnp.int32)
    dst = edge_index[1].astype(jnp.int32)
    pad = E_PAD - E
    srcp = jnp.concatenate([src, jnp.zeros((pad,), jnp.int32)]).reshape(
        NS, NCHUNK, CHUNK)
    dstp = jnp.concatenate([dst, jnp.full((pad,), N, jnp.int32)]).reshape(
        NS, NCHUNK, CHUNK)

    z64 = jnp.zeros((CHUNK, DQ), jnp.float32)
    z16 = jnp.zeros((CHUNK, 16), jnp.float32)
    o16 = jnp.ones((CHUNK, 16), jnp.float32)

    h, *hqs = _mlp_call(features, W1.T, b1.reshape(1, -1), W2.T,
                        b2.reshape(1, -1), W3.T, b3.reshape(1, -1))
    hq, h2q = hqs[:4], hqs[4:]

    sc1 = _build_sc1()
    (s1q0, s1q1, s1q2, s1q3, s2q0, s2q1, s2q2, s2q3, dga, dgb) = sc1(
        srcp, dstp, z64, o16, z16, *hq, *h2q)

    c16, *u1q = _stage2_call((s1q0, s1q1, s1q2, s1q3),
                             (s2q0, s2q1, s2q2, s2q3), dga, dgb, h,
                             Wf.reshape(1, -1))
    c16p = jnp.concatenate([c16, jnp.zeros((NP16 - N, 16), jnp.float32)])

    sc2 = _build_sc2()
    (oq0, oq1, oq2, oq3, csa, csb) = sc2(srcp, dstp, z64, z16, c16p, *u1q)

    u2q = _stage3_call((oq0, oq1, oq2, oq3), csa, csb, dga, dgb, c16, h)

    sc3 = _build_sc3()
    pq = sc3(srcp, dstp, z64, c16p, *u2q)

    return _stage4_call(pq, csa, csb, dga, dgb, c16, h)
